# Initial kernel scaffold; baseline (speedup 1.0000x reference)
import functools

import jax
import jax.numpy as jnp
from jax import lax
from jax.experimental import pallas as pl
from jax.experimental.pallas import tpu as pltpu
from jax.experimental.pallas import tpu_sc as plsc

_Q = 512
_K = 100000
_D = 256
_KNN = 16
_H = 4
_CTX = 256
_W = 512
_B = 196
_KP = _B * _W
_BPAD = 208
_NTILES = 32
_QPT = _Q // _NTILES


def _keys_body(p_ref, mem_ref, keys_ref, bmin_ref):
    b = pl.program_id(0)
    mem = mem_ref[...]
    pm = lax.dot_general(p_ref[...], mem, (((1,), (1,)), ((), ())),
                         preferred_element_type=jnp.float32)
    ones = jnp.ones((1, mem.shape[1]), jnp.float32)
    m2 = lax.dot_general(ones, mem * mem, (((1,), (1,)), ((), ())),
                         preferred_element_type=jnp.float32)
    keys = m2 - 2.0 * pm
    col = b * _W + lax.broadcasted_iota(jnp.int32, keys.shape, 1)
    keys = jnp.where(col < _K, keys, jnp.inf)
    keys_ref[...] = keys
    bmin_ref[...] = jnp.min(keys, axis=1, keepdims=True)


def _keys_stage(p_t, memory):
    return pl.pallas_call(
        _keys_body,
        grid=(_B,),
        in_specs=[
            pl.BlockSpec((_Q, _D), lambda b: (0, 0)),
            pl.BlockSpec((_W, _D), lambda b: (b, 0)),
        ],
        out_specs=[
            pl.BlockSpec((_Q, _W), lambda b: (0, b)),
            pl.BlockSpec((_Q, 1), lambda b: (0, b)),
        ],
        out_shape=[
            jax.ShapeDtypeStruct((_Q, _KP), jnp.float32),
            jax.ShapeDtypeStruct((_Q, _B), jnp.float32),
        ],
    )(p_t, memory)


def _merge16(best_v, best_p, v, p):
    sv, sp = plsc.sort_key_val(v, p)
    rv = lax.rev(sv, (0,))
    rp = lax.rev(sp, (0,))
    m = best_v <= rv
    lo_v = jnp.where(m, best_v, rv)
    lo_p = jnp.where(m, best_p, rp)
    return plsc.sort_key_val(lo_v, lo_p)


def _retrieve_stage(bmins_pad, keys_flat, memory):
    mesh = plsc.VectorSubcoreMesh(core_axis_name="c", subcore_axis_name="s")

    @functools.partial(
        pl.kernel,
        mesh=mesh,
        out_type=jax.ShapeDtypeStruct((_Q * _KNN, _D), jnp.float32),
        scratch_types=[
            pltpu.VMEM((_BPAD,), jnp.float32),
            pltpu.VMEM((_KNN,), jnp.int32),
            pltpu.VMEM((_KNN, _W), jnp.float32),
            pltpu.VMEM((_KNN,), jnp.int32),
            pltpu.VMEM((_KNN,), jnp.int32),
            pltpu.VMEM((_KNN, _D), jnp.float32),
            pltpu.SemaphoreType.DMA,
        ],
    )
    def sc_kernel(bmins_hbm, keys_hbm, mem_hbm, out_hbm,
                  brow_v, cand_v, ckeys_v, bpos_v, kidx_v, rows_v, sem):
        wid = lax.axis_index("s") * 2 + lax.axis_index("c")
        iota16 = lax.iota(jnp.int32, 16)
        inf16 = jnp.full((16,), jnp.inf, jnp.float32)
        zero16 = jnp.zeros((16,), jnp.int32)

        @pl.loop(0, _QPT)
        def _per_query(qi):
            q = wid * _QPT + qi
            pltpu.sync_copy(bmins_hbm.at[q], brow_v)

            def sel_step(c, carry):
                bv, bp = carry
                v = brow_v[pl.ds(c * 16, 16)]
                pos = c * 16 + iota16
                return _merge16(bv, bp, v, pos)

            bv, bp = lax.fori_loop(0, _BPAD // 16, sel_step, (inf16, zero16))
            bpos_v[...] = bp
            cand_v[...] = q * _B + bp

            pltpu.async_copy(keys_hbm.at[cand_v], ckeys_v, sem).wait()

            def scan_block(blk, carry0):
                def chunk_step(j, carry):
                    bv2, bp2 = carry
                    v = ckeys_v[blk, pl.ds(j * 16, 16)]
                    thr = jnp.max(bv2)
                    hit = jnp.any(v < thr)
                    pos = blk * _W + j * 16 + iota16

                    def do(op):
                        return _merge16(*op)

                    def skip(op):
                        return op[0], op[1]

                    return lax.cond(hit, do, skip, (bv2, bp2, v, pos))

                return lax.fori_loop(0, _W // 16, chunk_step, carry0)

            bv2, bp2 = lax.fori_loop(0, _KNN, scan_block, (inf16, zero16))

            slot = lax.shift_right_logical(bp2, 9)
            off = jnp.bitwise_and(bp2, _W - 1)
            blkid = plsc.load_gather(bpos_v, [slot])
            kidx_v[...] = blkid * _W + off

            pltpu.async_copy(mem_hbm.at[kidx_v], rows_v, sem).wait()
            pltpu.sync_copy(rows_v, out_hbm.at[pl.ds(q * _KNN, _KNN)])

    return sc_kernel(bmins_pad, keys_flat, memory)


def _attn_body(p_ref, ret_ref, wq_ref, bq_ref, wk_ref, bk_ref, wv_ref, bv_ref,
               wo_ref, bo_ref, w1_ref, b1_ref, w2_ref, b2_ref, out_ref):
    nt = (((1,), (1,)), ((), ()))
    p = p_ref[...]
    ret = ret_ref[...]
    q = lax.dot_general(p, wq_ref[...], nt,
                        preferred_element_type=jnp.float32) + bq_ref[...]
    kk = lax.dot_general(ret, wk_ref[...], nt,
                         preferred_element_type=jnp.float32) + bk_ref[...]
    vv = lax.dot_general(ret, wv_ref[...], nt,
                         preferred_element_type=jnp.float32) + bv_ref[...]
    dh = _D // _H
    ctxs = []
    for h in range(_H):
        qh = q[:, h * dh:(h + 1) * dh]
        kh = kk[:, h * dh:(h + 1) * dh].reshape(_Q, _KNN, dh)
        vh = vv[:, h * dh:(h + 1) * dh].reshape(_Q, _KNN, dh)
        s = jnp.sum(qh[:, None, :] * kh, axis=2) * (1.0 / (dh ** 0.5))
        s = s - jnp.max(s, axis=1, keepdims=True)
        e = jnp.exp(s)
        a = e / jnp.sum(e, axis=1, keepdims=True)
        ctxs.append(jnp.sum(a[:, :, None] * vh, axis=1))
    ctx = jnp.concatenate(ctxs, axis=1)
    att = lax.dot_general(ctx, wo_ref[...], nt,
                          preferred_element_type=jnp.float32) + bo_ref[...]
    h1 = jnp.maximum(
        lax.dot_general(att, w1_ref[...], nt,
                        preferred_element_type=jnp.float32) + b1_ref[...], 0.0)
    out_ref[...] = lax.dot_general(
        h1, w2_ref[...], nt,
        preferred_element_type=jnp.float32) + b2_ref[...]


def _attn_stage(p_t, retrieved, Wq, bq, Wk, bk, Wv, bv, Wo, bo, W1, b1, W2, b2):
    return pl.pallas_call(
        _attn_body,
        out_shape=jax.ShapeDtypeStruct((_Q, _CTX), jnp.float32),
    )(p_t, retrieved, Wq, bq.reshape(1, -1), Wk, bk.reshape(1, -1),
      Wv, bv.reshape(1, -1), Wo, bo.reshape(1, -1), W1, b1.reshape(1, -1),
      W2, b2.reshape(1, -1))


def kernel(p_t, memory, Wq, bq, Wk, bk, Wv, bv, Wo, bo, W1, b1, W2, b2):
    keys, bmins = _keys_stage(p_t, memory)
    bmins_pad = jnp.pad(bmins, ((0, 0), (0, _BPAD - _B)),
                        constant_values=jnp.inf)
    keys_flat = keys.reshape(_Q * _B, _W)
    retrieved = _retrieve_stage(bmins_pad, keys_flat, memory)
    return _attn_stage(p_t, retrieved, Wq, bq, Wk, bk, Wv, bv, Wo, bo,
                       W1, b1, W2, b2)


# speedup vs baseline: 2.7943x; 2.7943x over previous
import dataclasses
import functools

import jax
import jax.numpy as jnp
from jax import lax
from jax.experimental import pallas as pl
from jax.experimental.pallas import tpu as pltpu
from jax.experimental.pallas import tpu_sc as plsc

_Q = 512
_K = 100000
_D = 256
_KNN = 16
_H = 4
_CTX = 256
_W = 512
_B = 196
_KP = _B * _W
_BPAD = 208
_NTILES = 32
_QPT = _Q // _NTILES


def _keys_body(p_ref, mem_ref, m2_ref, p2_ref, keys_ref, bmin_ref):
    b = pl.program_id(0)
    mem = mem_ref[...]
    pm = lax.dot_general(p_ref[...], mem, (((1,), (1,)), ((), ())),
                         preferred_element_type=jnp.float32)
    keys = (p2_ref[...] + m2_ref[...]) - 2.0 * pm
    col = b * _W + lax.broadcasted_iota(jnp.int32, keys.shape, 1)
    keys = jnp.where(col < _K, keys, jnp.inf)
    keys_ref[...] = keys
    bmin_ref[...] = jnp.min(keys, axis=1, keepdims=True).reshape(1, 1, _Q)


def _keys_stage(p_t, memory, m2, p2):
    return pl.pallas_call(
        _keys_body,
        grid=(_B,),
        in_specs=[
            pl.BlockSpec((_Q, _D), lambda b: (0, 0)),
            pl.BlockSpec((_W, _D), lambda b: (b, 0)),
            pl.BlockSpec((1, _W), lambda b: (0, b)),
            pl.BlockSpec((_Q, 1), lambda b: (0, 0)),
        ],
        out_specs=[
            pl.BlockSpec((_Q, _W), lambda b: (0, b)),
            pl.BlockSpec((1, 1, _Q), lambda b: (b, 0, 0)),
        ],
        out_shape=[
            jax.ShapeDtypeStruct((_Q, _KP), jnp.float32),
            jax.ShapeDtypeStruct((_B, 1, _Q), jnp.float32),
        ],
    )(p_t, memory, m2, p2)


def _merge16(best_v, best_p, v, p):
    sv, sp = plsc.sort_key_val(v, p)
    rv = lax.rev(sv, (0,))
    rp = lax.rev(sp, (0,))
    m = best_v <= rv
    lo_v = jnp.where(m, best_v, rv)
    lo_p = jnp.where(m, best_p, rp)
    out_v, out_p = plsc.sort_key_val(lo_v, lo_p)
    return out_v, out_p


def _retrieve_stage(bmins_pad, keys_flat, memory):
    mesh = plsc.VectorSubcoreMesh(core_axis_name="c", subcore_axis_name="s")
    cp = pltpu.CompilerParams()
    if "needs_layout_passes" in pltpu.CompilerParams.__dataclass_fields__:
        cp = dataclasses.replace(cp, needs_layout_passes=False)

    @functools.partial(
        pl.kernel,
        mesh=mesh,
        compiler_params=cp,
        out_type=jax.ShapeDtypeStruct((_Q * _KNN, _D), jnp.float32),
        scratch_types=[
            pltpu.VMEM((_BPAD,), jnp.float32),
            pltpu.VMEM((_KNN,), jnp.int32),
            pltpu.VMEM((_KNN, _W), jnp.float32),
            pltpu.VMEM((_KNN,), jnp.int32),
            pltpu.VMEM((_KNN,), jnp.int32),
            pltpu.VMEM((_KNN, _D), jnp.float32),
            pltpu.SemaphoreType.DMA,
        ],
    )
    def sc_kernel(bmins_hbm, keys_hbm, mem_hbm, out_hbm,
                  brow_v, cand_v, ckeys_v, bpos_v, kidx_v, rows_v, sem):
        wid = lax.axis_index("s") * 2 + lax.axis_index("c")
        iota16 = lax.iota(jnp.int32, 16)
        inf16 = jnp.full((16,), jnp.inf, jnp.float32)
        zero16 = jnp.zeros((16,), jnp.int32)

        @pl.loop(0, _QPT)
        def _per_query(qi):
            q = wid * _QPT + qi
            pltpu.sync_copy(bmins_hbm.at[q], brow_v)

            def sel_step(c, carry):
                bv, bp = carry
                v = brow_v[pl.ds(c * 16, 16)]
                pos = c * 16 + iota16
                return _merge16(bv, bp, v, pos)

            bv, bp = lax.fori_loop(0, _BPAD // 16, sel_step, (inf16, zero16))
            bpos_v[...] = bp
            cand_v[...] = q * _B + bp

            pltpu.async_copy(keys_hbm.at[cand_v], ckeys_v, sem).wait()

            def scan_block(blk, carry0):
                def chunk_step(j, carry):
                    bv2, bp2 = carry
                    v = ckeys_v[blk, pl.ds(j * 16, 16)]
                    thr = jnp.max(bv2)
                    hit = jnp.any(v < thr)
                    pos = blk * _W + j * 16 + iota16

                    def do(op):
                        return _merge16(*op)

                    def skip(op):
                        return op[0], op[1]

                    return lax.cond(hit, do, skip, (bv2, bp2, v, pos))

                return lax.fori_loop(0, _W // 16, chunk_step, carry0)

            bv2, bp2 = lax.fori_loop(0, _KNN, scan_block, (inf16, zero16))

            slot = lax.shift_right_logical(bp2, 9)
            off = jnp.bitwise_and(bp2, _W - 1)
            blkid = plsc.load_gather(bpos_v, [slot])
            kidx_v[...] = blkid * _W + off

            pltpu.async_copy(mem_hbm.at[kidx_v], rows_v, sem).wait()
            pltpu.sync_copy(rows_v, out_hbm.at[pl.ds(q * _KNN, _KNN)])

    return sc_kernel(bmins_pad, keys_flat, memory)


def _attn_body(p_ref, ret_ref, wq_ref, bq_ref, wk_ref, bk_ref, wv_ref, bv_ref,
               wo_ref, bo_ref, w1_ref, b1_ref, w2_ref, b2_ref, out_ref):
    nt = (((1,), (1,)), ((), ()))
    p = p_ref[...]
    ret = ret_ref[...]
    q = lax.dot_general(p, wq_ref[...], nt,
                        preferred_element_type=jnp.float32) + bq_ref[...]
    kk = lax.dot_general(ret, wk_ref[...], nt,
                         preferred_element_type=jnp.float32) + bk_ref[...]
    vv = lax.dot_general(ret, wv_ref[...], nt,
                         preferred_element_type=jnp.float32) + bv_ref[...]
    dh = _D // _H
    ctxs = []
    for h in range(_H):
        qh = q[:, h * dh:(h + 1) * dh]
        kh = kk[:, h * dh:(h + 1) * dh].reshape(_Q, _KNN, dh)
        vh = vv[:, h * dh:(h + 1) * dh].reshape(_Q, _KNN, dh)
        s = jnp.sum(qh[:, None, :] * kh, axis=2) * (1.0 / (dh ** 0.5))
        s = s - jnp.max(s, axis=1, keepdims=True)
        e = jnp.exp(s)
        a = e / jnp.sum(e, axis=1, keepdims=True)
        ctxs.append(jnp.sum(a[:, :, None] * vh, axis=1))
    ctx = jnp.concatenate(ctxs, axis=1)
    att = lax.dot_general(ctx, wo_ref[...], nt,
                          preferred_element_type=jnp.float32) + bo_ref[...]
    h1 = jnp.maximum(
        lax.dot_general(att, w1_ref[...], nt,
                        preferred_element_type=jnp.float32) + b1_ref[...], 0.0)
    out_ref[...] = lax.dot_general(
        h1, w2_ref[...], nt,
        preferred_element_type=jnp.float32) + b2_ref[...]


def _attn_stage(p_t, retrieved, Wq, bq, Wk, bk, Wv, bv, Wo, bo, W1, b1, W2, b2):
    return pl.pallas_call(
        _attn_body,
        out_shape=jax.ShapeDtypeStruct((_Q, _CTX), jnp.float32),
    )(p_t, retrieved, Wq, bq.reshape(1, -1), Wk, bk.reshape(1, -1),
      Wv, bv.reshape(1, -1), Wo, bo.reshape(1, -1), W1, b1.reshape(1, -1),
      W2, b2.reshape(1, -1))


def kernel(p_t, memory, Wq, bq, Wk, bk, Wv, bv, Wo, bo, W1, b1, W2, b2):
    m2 = jnp.sum(memory * memory, axis=-1)[None, :]
    p2 = jnp.sum(p_t * p_t, axis=-1, keepdims=True)
    keys, bmins_t = _keys_stage(p_t, memory, m2, p2)
    bmins = bmins_t.reshape(_B, _Q).T
    bmins_pad = jnp.pad(bmins, ((0, 0), (0, _BPAD - _B)),
                        constant_values=jnp.inf)
    keys_flat = keys.reshape(_Q * _B, _W)
    retrieved = _retrieve_stage(bmins_pad, keys_flat, memory)
    return _attn_stage(p_t, retrieved, Wq, bq, Wk, bk, Wv, bv, Wo, bo,
                       W1, b1, W2, b2)
